# baseline (speedup 1.0000x reference)
import functools

import jax
import jax.numpy as jnp
from jax import lax
from jax.experimental import pallas as pl
from jax.experimental.pallas import tpu as pltpu
from jax.experimental.pallas import tpu_sc as plsc

D = 32
B = 16384
L = 50
NC = 2
NS = 16
NW = NC * NS
BPW = B // NW
CH = 32
ROWS = CH * L
NCHUNK = BPW // CH
G = 80
NG = ROWS // G

_INV_PERM = tuple(
    (c // 2) if c % 2 == 0 else 16 + c // 2 for c in range(D))


def _embed_bag_body(idx_hbm, len_hbm, table_hbm, out_hbm,
                    idx_v, buf_v, out_v, len_v, sem):
    wid = lax.axis_index("s") * NC + lax.axis_index("c")
    base_b = wid * BPW

    pltpu.sync_copy(len_hbm.at[pl.ds(base_b * 1, BPW)], len_v.at[pl.ds(0, BPW)])

    def chunk_body(c, carry):
        flat_base = pl.multiple_of((base_b + c * CH) * L, 8)
        pltpu.sync_copy(idx_hbm.at[pl.ds(flat_base, ROWS)], idx_v)

        copies = []
        for j in range(NG):
            copies.append(pltpu.async_copy(
                table_hbm.at[idx_v.at[pl.ds(j * G, G)]],
                buf_v.at[pl.ds(j * G, G)],
                sem))
        for cp in copies:
            cp.wait()

        def batch_body(b, bcarry):
            r0 = b * L
            a0, b0 = plsc.unpack(buf_v[r0], format=plsc.PackFormat.INTERLEAVED)
            acc0, acc1 = a0, b0
            for l in range(1, L):
                a, bb = plsc.unpack(buf_v[r0 + l],
                                    format=plsc.PackFormat.INTERLEAVED)
                acc0 = acc0 + a
                acc1 = acc1 + bb
            lnv = len_v[pl.ds(c * CH + b, 16)]
            lf = jnp.maximum(lnv[0], 1).astype(jnp.float32)
            out_v[b, pl.ds(0, 16)] = acc0 / lf
            out_v[b, pl.ds(16, 16)] = acc1 / lf
            return bcarry

        lax.fori_loop(0, CH, batch_body, 0)

        out_base = pl.multiple_of(base_b + c * CH, 8)
        pltpu.sync_copy(out_v, out_hbm.at[pl.ds(out_base, CH)])
        return carry

    lax.fori_loop(0, NCHUNK, chunk_body, 0)


@jax.jit
def _embed_bag(idx_flat, len_flat, table_bf):
    mesh = plsc.VectorSubcoreMesh(core_axis_name="c", subcore_axis_name="s")
    return pl.kernel(
        _embed_bag_body,
        out_type=jax.ShapeDtypeStruct((B, D), jnp.float32),
        mesh=mesh,
        compiler_params=pltpu.CompilerParams(
            use_tc_tiling_on_sc=False, needs_layout_passes=False),
        scratch_types=[
            pltpu.VMEM((ROWS,), jnp.int32),
            pltpu.VMEM((ROWS, D), jnp.bfloat16),
            pltpu.VMEM((CH, D), jnp.float32),
            pltpu.VMEM((BPW + 16,), jnp.int32),
            pltpu.SemaphoreType.DMA,
        ],
    )(idx_flat, len_flat, table_bf)


def kernel(kw_indices, kw_lengths, embedding_weight):
    idx_flat = kw_indices.reshape(-1).astype(jnp.int32)
    len_flat = kw_lengths.reshape(-1).astype(jnp.int32)
    table_bf = embedding_weight.astype(jnp.bfloat16)
    out = _embed_bag(idx_flat, len_flat, table_bf)
    return jnp.take(out, jnp.array(_INV_PERM, jnp.int32), axis=1)
